# baseline (speedup 1.0000x reference)
import functools

import jax
import jax.numpy as jnp
import numpy as np
from jax import lax
from jax.experimental import pallas as pl
from jax.experimental.pallas import tpu as pltpu
from jax.experimental.pallas import tpu_sc as plsc

N = 10000
E = 320000
EPS = 1e-4

NW = 32
E_PER_W = E // NW
CHUNK = 80
NCHUNK_G = E_PER_W // CHUNK
E_PER_T = E // 16
NCHUNK_S = E_PER_T // CHUNK
ROWS_A = 624
ROWS_TAIL = N - 16 * ROWS_A

_SC_MESH = dict(core_axis_name="c", subcore_axis_name="s")
_MASK_HI = np.uint32(0xFFFF0000)
_HALF = np.uint32(0x8000)


def _pack_bf16(x):
    ua = lax.bitcast_convert_type(x[:, 0:64], jnp.uint32)
    ua = (ua + _HALF) & _MASK_HI
    ub = lax.bitcast_convert_type(x[:, 64:128], jnp.uint32)
    ub = (ub + _HALF) >> 16
    return lax.bitcast_convert_type(ua | ub, jnp.float32)


def _unpack_bf16(w):
    u = lax.bitcast_convert_type(w, jnp.uint32)
    hi = lax.bitcast_convert_type(u & _MASK_HI, jnp.float32)
    lo = lax.bitcast_convert_type(u << 16, jnp.float32)
    return jnp.concatenate([hi, lo], axis=1)


def _prep_body(s_ref, vt_ref, wd_ref, ws_ref, td_ref, ts_ref):
    sblk = s_ref[...]
    vt = vt_ref[...]
    zpad = jnp.zeros((sblk.shape[0], 16), jnp.float32)
    for w_ref, out_ref in ((wd_ref, td_ref), (ws_ref, ts_ref)):
        proj = jnp.dot(sblk, w_ref[...], preferred_element_type=jnp.float32)
        out_ref[:, 0:64] = _pack_bf16(proj)
        out_ref[:, 64:112] = vt
        out_ref[:, 112:128] = zpad


def _prep(s, vt, wdT, wsT):
    bn = 2000
    return pl.pallas_call(
        _prep_body,
        grid=(N // bn,),
        in_specs=[
            pl.BlockSpec((bn, 128), lambda i: (i, 0)),
            pl.BlockSpec((bn, 48), lambda i: (i, 0)),
            pl.BlockSpec((128, 128), lambda i: (0, 0)),
            pl.BlockSpec((128, 128), lambda i: (0, 0)),
        ],
        out_specs=[
            pl.BlockSpec((bn, 128), lambda i: (i, 0)),
            pl.BlockSpec((bn, 128), lambda i: (i, 0)),
        ],
        out_shape=[
            jax.ShapeDtypeStruct((N, 128), jnp.float32),
            jax.ShapeDtypeStruct((N, 128), jnp.float32),
        ],
    )(s, vt, wdT, wsT)


def _sc_gather_body(td_hbm, ts_hbm, idxd_hbm, idxs_hbm, gd_hbm, gs_hbm,
                    idxd_v, idxs_v, bufd, bufs, semd, sems):
    cid = lax.axis_index("c")
    sid = lax.axis_index("s")
    wid = cid * 16 + sid
    pltpu.sync_copy(idxd_hbm.at[wid], idxd_v)
    pltpu.sync_copy(idxs_hbm.at[wid], idxs_v)

    def body(j, carry):
        base = wid * E_PER_W + j * CHUNK
        cpd = pltpu.async_copy(td_hbm.at[idxd_v.at[j]], bufd, semd)
        cps = pltpu.async_copy(ts_hbm.at[idxs_v.at[j]], bufs, sems)
        cpd.wait()
        pltpu.sync_copy(bufd, gd_hbm.at[pl.ds(base, CHUNK)])
        cps.wait()
        pltpu.sync_copy(bufs, gs_hbm.at[pl.ds(base, CHUNK)])
        return carry

    lax.fori_loop(0, NCHUNK_G, body, 0)


def _sc_gather(td, ts, idxd3, idxs3):
    k = functools.partial(
        pl.kernel,
        out_type=(
            jax.ShapeDtypeStruct((E, 128), jnp.float32),
            jax.ShapeDtypeStruct((E, 128), jnp.float32),
        ),
        mesh=plsc.VectorSubcoreMesh(**_SC_MESH),
        scratch_types=[
            pltpu.VMEM((NCHUNK_G, CHUNK), jnp.int32),
            pltpu.VMEM((NCHUNK_G, CHUNK), jnp.int32),
            pltpu.VMEM((CHUNK, 128), jnp.float32),
            pltpu.VMEM((CHUNK, 128), jnp.float32),
            pltpu.SemaphoreType.DMA,
            pltpu.SemaphoreType.DMA,
        ],
    )(_sc_gather_body)
    return k(td, ts, idxd3, idxs3)


def _edge_body(gd_ref, gs_ref, es_ref, ev_ref,
               a_ref, b_ref, c_ref, m0_ref, w0_ref, b0_ref,
               wh1T_ref, wmu1T_ref, w1_ref, b1_ref,
               wh2T_ref, wmu2T_ref, w2_ref, b2_ref,
               whaT_ref, wa_ref, ba_ref,
               ms_ref, mv_ref):
    f32 = jnp.float32
    bf16 = jnp.bfloat16
    gd = gd_ref[...]
    gs = gs_ref[...]
    ev = ev_ref[...]
    be = gd.shape[0]

    def vdims(x):
        return jnp.concatenate(
            [x[:, 64 + 16 * d:64 + 16 * (d + 1)] for d in range(3)], axis=0)

    def n3(x):
        return jnp.sqrt(x[0:be] ** 2 + x[be:2 * be] ** 2 + x[2 * be:3 * be] ** 2)

    def g3(g):
        return jnp.concatenate([g, g, g], axis=0)

    ev3 = jnp.concatenate([ev[:, d:d + 1] for d in range(3)], axis=0)
    Vh = (jnp.dot(vdims(gd), a_ref[...], preferred_element_type=f32)
          + jnp.dot(vdims(gs), b_ref[...], preferred_element_type=f32)
          + ev3 * c_ref[...])
    sh = jnp.maximum(n3(Vh), EPS)
    x0 = jnp.concatenate([es_ref[...], sh], axis=1).astype(bf16)
    sm = (_unpack_bf16(gd[:, 0:64]) + _unpack_bf16(gs[:, 0:64])
          + jnp.dot(x0, w0_ref[...], preferred_element_type=f32) + b0_ref[...])
    se = jax.nn.relu(sm)
    Vmu = jnp.dot(Vh, m0_ref[...], preferred_element_type=f32)
    gate = jax.nn.sigmoid(jnp.maximum(n3(Vmu), EPS))
    Vg = g3(gate) * Vmu

    for whT, wmuT, w, bb, is_last in (
        (wh1T_ref, wmu1T_ref, w1_ref, b1_ref, False),
        (wh2T_ref, wmu2T_ref, w2_ref, b2_ref, True),
    ):
        Vh = jnp.dot(Vg, whT[...], preferred_element_type=f32)
        sh = jnp.maximum(n3(Vh), EPS)
        x = jnp.concatenate([se, sh], axis=1).astype(bf16)
        sm = jnp.dot(x, w[...], preferred_element_type=f32) + bb[...]
        se = sm if is_last else jax.nn.relu(sm)
        Vmu = jnp.dot(Vh, wmuT[...], preferred_element_type=f32)
        vn = jnp.maximum(n3(Vmu), EPS)
        gate = vn if is_last else jax.nn.sigmoid(vn)
        Vg = g3(gate) * Vmu

    Vha = jnp.dot(Vg, whaT_ref[...], preferred_element_type=f32)
    sha = jnp.maximum(n3(Vha), EPS)
    xa = jnp.concatenate([se, sha], axis=1).astype(bf16)
    att = jax.nn.sigmoid(
        jnp.dot(xa, wa_ref[...], preferred_element_type=f32) + ba_ref[...])

    ms_ref[...] = att * se
    for d in range(3):
        mv_ref[:, 16 * d:16 * (d + 1)] = att * Vg[d * be:(d + 1) * be]
    mv_ref[:, 48:128] = jnp.zeros((be, 80), f32)


def _edge(gd, gs, es, ev, wts):
    be = 4000
    full = lambda arr: pl.BlockSpec(arr.shape, lambda i: tuple(0 for _ in arr.shape))
    return pl.pallas_call(
        _edge_body,
        grid=(E // be,),
        in_specs=[
            pl.BlockSpec((be, 128), lambda i: (i, 0)),
            pl.BlockSpec((be, 128), lambda i: (i, 0)),
            pl.BlockSpec((be, 32), lambda i: (i, 0)),
            pl.BlockSpec((be, 3), lambda i: (i, 0)),
        ] + [full(w) for w in wts],
        out_specs=[
            pl.BlockSpec((be, 128), lambda i: (i, 0)),
            pl.BlockSpec((be, 128), lambda i: (i, 0)),
        ],
        out_shape=[
            jax.ShapeDtypeStruct((E, 128), jnp.float32),
            jax.ShapeDtypeStruct((E, 128), jnp.float32),
        ],
        compiler_params=pltpu.CompilerParams(
            dimension_semantics=("arbitrary",)),
    )(gd, gs, es, ev, *wts)


def _sc_scatter_body(ms_hbm, mv_hbm, idx_hbm, zeros_hbm, outs_hbm, outv_hbm,
                     acc, idx_v, buf, sem):
    cid = lax.axis_index("c")
    sid = lax.axis_index("s")
    base_r = sid * ROWS_A
    pltpu.sync_copy(zeros_hbm.at[pl.ds(base_r, ROWS_A)],
                    acc.at[pl.ds(base_r, ROWS_A)])

    @pl.when(sid == 0)
    def _():
        pltpu.sync_copy(zeros_hbm.at[pl.ds(16 * ROWS_A, ROWS_TAIL)],
                        acc.at[pl.ds(16 * ROWS_A, ROWS_TAIL)])

    pltpu.sync_copy(idx_hbm.at[sid], idx_v)
    plsc.subcore_barrier()

    def run(m_hbm):
        def body(j, carry):
            base = sid * E_PER_T + j * CHUNK
            pltpu.sync_copy(m_hbm.at[pl.ds(base, CHUNK)], buf)
            pltpu.sync_copy(buf, acc.at[idx_v.at[j]], add=True)
            return carry
        lax.fori_loop(0, NCHUNK_S, body, 0)

    @pl.when(cid == 0)
    def _():
        run(ms_hbm)

    @pl.when(cid == 1)
    def _():
        run(mv_hbm)

    plsc.subcore_barrier()

    def out(out_hbm):
        pltpu.sync_copy(acc.at[pl.ds(base_r, ROWS_A)],
                        out_hbm.at[pl.ds(base_r, ROWS_A)])

        @pl.when(sid == 0)
        def _():
            pltpu.sync_copy(acc.at[pl.ds(16 * ROWS_A, ROWS_TAIL)],
                            out_hbm.at[pl.ds(16 * ROWS_A, ROWS_TAIL)])

    @pl.when(cid == 0)
    def _():
        out(outs_hbm)

    @pl.when(cid == 1)
    def _():
        out(outv_hbm)


def _sc_scatter(ms, mv, idx3, zeros_nrow):
    k = functools.partial(
        pl.kernel,
        out_type=(
            jax.ShapeDtypeStruct((N, 128), jnp.float32),
            jax.ShapeDtypeStruct((N, 128), jnp.float32),
        ),
        mesh=plsc.VectorSubcoreMesh(**_SC_MESH),
        scratch_types=[
            pltpu.VMEM_SHARED((N, 128), jnp.float32),
            pltpu.VMEM((NCHUNK_S, CHUNK), jnp.int32),
            pltpu.VMEM((CHUNK, 128), jnp.float32),
            pltpu.SemaphoreType.DMA,
        ],
    )(_sc_scatter_body)
    return k(ms, mv, idx3, zeros_nrow)


def _node_body(s_ref, vt_ref, as_ref, av_ref,
               fwh0T_ref, fmu0T_ref, f0sT_ref, f0hT_ref, f0b_ref,
               fwh1T_ref, fmu1T_ref, f1sT_ref, f1hT_ref, f1b_ref,
               g0_ref, c0_ref, g1_ref, c1_ref,
               s2_ref, v2_ref):
    f32 = jnp.float32

    def ln(sx, Vx, g, b):
        mu = jnp.mean(sx, axis=-1, keepdims=True)
        var = jnp.mean((sx - mu) ** 2, axis=-1, keepdims=True)
        so = (sx - mu) * jax.lax.rsqrt(var + 1e-5) * g + b
        nrm = jnp.sqrt(jnp.sum(Vx[0] ** 2 + Vx[1] ** 2 + Vx[2] ** 2,
                               axis=-1, keepdims=True)) * 0.25
        nrm = jnp.maximum(nrm, 0.0031622776601683794)
        inv = 1.0 / nrm
        return so, [v * inv for v in Vx]

    s_in = s_ref[...] + as_ref[...]
    V_in = [vt_ref[:, 16 * d:16 * (d + 1)] + av_ref[:, 16 * d:16 * (d + 1)]
            for d in range(3)]
    s1, V1 = ln(s_in, V_in, g0_ref[...], c0_ref[...])

    se, Vg = s1, V1
    for whT, wmuT, wsT, whsT, bb, is_last in (
        (fwh0T_ref, fmu0T_ref, f0sT_ref, f0hT_ref, f0b_ref, False),
        (fwh1T_ref, fmu1T_ref, f1sT_ref, f1hT_ref, f1b_ref, True),
    ):
        Vh = [jnp.dot(v, whT[...], preferred_element_type=f32) for v in Vg]
        sh = jnp.maximum(jnp.sqrt(Vh[0] ** 2 + Vh[1] ** 2 + Vh[2] ** 2), EPS)
        sm = (jnp.dot(se, wsT[...], preferred_element_type=f32)
              + jnp.dot(sh, whsT[...], preferred_element_type=f32)
              + bb[...])
        se = sm if is_last else jax.nn.relu(sm)
        Vmu = [jnp.dot(v, wmuT[...], preferred_element_type=f32) for v in Vh]
        vn = jnp.maximum(jnp.sqrt(Vmu[0] ** 2 + Vmu[1] ** 2 + Vmu[2] ** 2), EPS)
        gate = vn if is_last else jax.nn.sigmoid(vn)
        Vg = [gate * v for v in Vmu]

    s2, V2 = ln(s1 + se, [a + b for a, b in zip(V1, Vg)], g1_ref[...], c1_ref[...])
    s2_ref[...] = s2
    for d in range(3):
        v2_ref[:, 16 * d:16 * (d + 1)] = V2[d]


def _node(s, vt, aggs, aggv, wts):
    bn = 2000
    full = lambda arr: pl.BlockSpec(arr.shape, lambda i: tuple(0 for _ in arr.shape))
    return pl.pallas_call(
        _node_body,
        grid=(N // bn,),
        in_specs=[
            pl.BlockSpec((bn, 128), lambda i: (i, 0)),
            pl.BlockSpec((bn, 48), lambda i: (i, 0)),
            pl.BlockSpec((bn, 128), lambda i: (i, 0)),
            pl.BlockSpec((bn, 128), lambda i: (i, 0)),
        ] + [full(w) for w in wts],
        out_specs=[
            pl.BlockSpec((bn, 128), lambda i: (i, 0)),
            pl.BlockSpec((bn, 48), lambda i: (i, 0)),
        ],
        out_shape=[
            jax.ShapeDtypeStruct((N, 128), jnp.float32),
            jax.ShapeDtypeStruct((N, 48), jnp.float32),
        ],
    )(s, vt, aggs, aggv, *wts)


def kernel(s, V, edge_index, edge_s, edge_v,
           eg0_Wh, eg0_Wmu, eg0_Wm_w, eg0_Wm_b,
           eg1_Wh, eg1_Wmu, eg1_Wm_w, eg1_Wm_b,
           eg2_Wh, eg2_Wmu, eg2_Wm_w, eg2_Wm_b,
           ff0_Wh, ff0_Wmu, ff0_Wm_w, ff0_Wm_b,
           ff1_Wh, ff1_Wmu, ff1_Wm_w, ff1_Wm_b,
           att_Wh, att_Wm_w, att_Wm_b,
           ln0_g, ln0_b, ln1_g, ln1_b):
    vt = V.transpose(0, 2, 1).reshape(N, 48)
    src = edge_index[0].astype(jnp.int32)
    dst = edge_index[1].astype(jnp.int32)
    idxd3 = dst.reshape(NW, NCHUNK_G, CHUNK)
    idxs3 = src.reshape(NW, NCHUNK_G, CHUNK)
    idxd_sc = dst.reshape(16, NCHUNK_S, CHUNK)
    ev = edge_v.reshape(E, 3)

    td, ts = _prep(s, vt, eg0_Wm_w[:, 0:128].T, eg0_Wm_w[:, 128:256].T)

    gd, gs = _sc_gather(td, ts, idxd3, idxs3)

    bf16 = jnp.bfloat16
    edge_wts = [
        eg0_Wh[:, 0:16].T, eg0_Wh[:, 16:32].T, eg0_Wh[:, 32].reshape(1, 33),
        eg0_Wmu.T, eg0_Wm_w[:, 256:321].T.astype(bf16),
        eg0_Wm_b.reshape(1, 128),
        eg1_Wh.T, eg1_Wmu.T, eg1_Wm_w.T.astype(bf16),
        eg1_Wm_b.reshape(1, 128),
        eg2_Wh.T, eg2_Wmu.T, eg2_Wm_w.T.astype(bf16),
        eg2_Wm_b.reshape(1, 128),
        att_Wh.T, att_Wm_w.T.astype(bf16), att_Wm_b.reshape(1, 1),
    ]
    ms, mv = _edge(gd, gs, edge_s, ev, edge_wts)

    zeros_nrow = jnp.zeros((N, 128), jnp.float32)
    aggs, aggv = _sc_scatter(ms, mv, idxd_sc, zeros_nrow)

    node_wts = [
        ff0_Wh.T, ff0_Wmu.T, ff0_Wm_w[:, 0:128].T, ff0_Wm_w[:, 128:144].T,
        ff0_Wm_b.reshape(1, 128),
        ff1_Wh.T, ff1_Wmu.T, ff1_Wm_w[:, 0:128].T, ff1_Wm_w[:, 128:144].T,
        ff1_Wm_b.reshape(1, 128),
        ln0_g.reshape(1, 128), ln0_b.reshape(1, 128),
        ln1_g.reshape(1, 128), ln1_b.reshape(1, 128),
    ]
    s2, v2t = _node(s, vt, aggs, aggv, node_wts)
    V2 = v2t.reshape(N, 3, 16).transpose(0, 2, 1)
    return s2, V2


# speedup vs baseline: 1.6676x; 1.6676x over previous
import functools

import jax
import jax.numpy as jnp
import numpy as np
from jax import lax
from jax.experimental import pallas as pl
from jax.experimental.pallas import tpu as pltpu
from jax.experimental.pallas import tpu_sc as plsc

N = 10000
E = 320000
EPS = 1e-4

NW = 32
E_PER_W = E // NW
CHUNK = 80
NCHUNK_G = E_PER_W // CHUNK
E_PER_T = E // 16
NCHUNK_S = E_PER_T // CHUNK
ROWS_A = 624
ROWS_TAIL = N - 16 * ROWS_A

_SC_MESH = dict(core_axis_name="c", subcore_axis_name="s")
_MASK_HI = np.uint32(0xFFFF0000)
_HALF = np.uint32(0x8000)


def _pack_bf16(x):
    ua = lax.bitcast_convert_type(x[:, 0:64], jnp.uint32)
    ua = (ua + _HALF) & _MASK_HI
    ub = lax.bitcast_convert_type(x[:, 64:128], jnp.uint32)
    ub = (ub + _HALF) >> 16
    return lax.bitcast_convert_type(ua | ub, jnp.float32)


def _unpack_bf16(w):
    u = lax.bitcast_convert_type(w, jnp.uint32)
    hi = lax.bitcast_convert_type(u & _MASK_HI, jnp.float32)
    lo = lax.bitcast_convert_type(u << 16, jnp.float32)
    return jnp.concatenate([hi, lo], axis=1)


def _prep_body(s_ref, vt_ref, wd_ref, ws_ref, td_ref, ts_ref):
    sblk = s_ref[...]
    vt = vt_ref[...]
    zpad = jnp.zeros((sblk.shape[0], 16), jnp.float32)
    for w_ref, out_ref in ((wd_ref, td_ref), (ws_ref, ts_ref)):
        proj = jnp.dot(sblk, w_ref[...], preferred_element_type=jnp.float32)
        out_ref[:, 0:64] = _pack_bf16(proj)
        out_ref[:, 64:112] = vt
        out_ref[:, 112:128] = zpad


def _prep(s, vt, wdT, wsT):
    bn = 2000
    return pl.pallas_call(
        _prep_body,
        grid=(N // bn,),
        in_specs=[
            pl.BlockSpec((bn, 128), lambda i: (i, 0)),
            pl.BlockSpec((bn, 48), lambda i: (i, 0)),
            pl.BlockSpec((128, 128), lambda i: (0, 0)),
            pl.BlockSpec((128, 128), lambda i: (0, 0)),
        ],
        out_specs=[
            pl.BlockSpec((bn, 128), lambda i: (i, 0)),
            pl.BlockSpec((bn, 128), lambda i: (i, 0)),
        ],
        out_shape=[
            jax.ShapeDtypeStruct((N, 128), jnp.float32),
            jax.ShapeDtypeStruct((N, 128), jnp.float32),
        ],
    )(s, vt, wdT, wsT)


def _sc_gather_body(td_hbm, ts_hbm, idxd_hbm, idxs_hbm, gd_hbm, gs_hbm,
                    idxd_v, idxs_v, bufd, bufs, semd, sems):
    cid = lax.axis_index("c")
    sid = lax.axis_index("s")
    wid = cid * 16 + sid
    pltpu.sync_copy(idxd_hbm.at[wid], idxd_v)
    pltpu.sync_copy(idxs_hbm.at[wid], idxs_v)

    def body(j, carry):
        base = wid * E_PER_W + j * CHUNK
        cpd = pltpu.async_copy(td_hbm.at[idxd_v.at[j]], bufd, semd)
        cps = pltpu.async_copy(ts_hbm.at[idxs_v.at[j]], bufs, sems)
        cpd.wait()
        pltpu.sync_copy(bufd, gd_hbm.at[pl.ds(base, CHUNK)])
        cps.wait()
        pltpu.sync_copy(bufs, gs_hbm.at[pl.ds(base, CHUNK)])
        return carry

    lax.fori_loop(0, NCHUNK_G, body, 0)


def _sc_gather(td, ts, idxd3, idxs3):
    k = functools.partial(
        pl.kernel,
        out_type=(
            jax.ShapeDtypeStruct((E, 128), jnp.float32),
            jax.ShapeDtypeStruct((E, 128), jnp.float32),
        ),
        mesh=plsc.VectorSubcoreMesh(**_SC_MESH),
        scratch_types=[
            pltpu.VMEM((NCHUNK_G, CHUNK), jnp.int32),
            pltpu.VMEM((NCHUNK_G, CHUNK), jnp.int32),
            pltpu.VMEM((CHUNK, 128), jnp.float32),
            pltpu.VMEM((CHUNK, 128), jnp.float32),
            pltpu.SemaphoreType.DMA,
            pltpu.SemaphoreType.DMA,
        ],
    )(_sc_gather_body)
    return k(td, ts, idxd3, idxs3)


_EPS2 = float(np.float32(EPS) * np.float32(EPS))


def _edge_body(gd_ref, gs_ref, es_ref, ev_ref,
               wc0_ref, s0_ref, g0_ref, w0_ref, b0_ref,
               wc1_ref, s1_ref, g1_ref, w1_ref, b1_ref,
               wc2_ref, w2_ref, b2_ref,
               wha_ref, s16_ref, wa_ref, ba_ref,
               ms_ref, mv_ref):
    f32 = jnp.float32
    bf16 = jnp.bfloat16
    gd = gd_ref[...]
    gs = gs_ref[...]
    be = gd.shape[0]

    def rnorm(p, s_ref):
        xc = jnp.maximum(
            jnp.dot(p * p, s_ref[...], preferred_element_type=f32), _EPS2)
        return xc * lax.rsqrt(xc)

    x0v = jnp.concatenate([gd[:, 64:112], gs[:, 64:112], ev_ref[...]], axis=1)
    P = jnp.dot(x0v, wc0_ref[...], preferred_element_type=f32)
    r = rnorm(P, s0_ref)
    x0 = jnp.concatenate([es_ref[...], r], axis=1).astype(bf16)
    sm = (_unpack_bf16(gd[:, 0:64]) + _unpack_bf16(gs[:, 0:64])
          + jnp.dot(x0, w0_ref[...], preferred_element_type=f32) + b0_ref[...])
    se = jax.nn.relu(sm)
    gate = jax.nn.sigmoid(jnp.dot(r, g0_ref[...], preferred_element_type=f32))
    Vg = gate * P[:, 0:48]

    P = jnp.dot(Vg, wc1_ref[...], preferred_element_type=f32)
    r = rnorm(P, s1_ref)
    x = jnp.concatenate([se, r], axis=1).astype(bf16)
    se = jax.nn.relu(
        jnp.dot(x, w1_ref[...], preferred_element_type=f32) + b1_ref[...])
    gate = jax.nn.sigmoid(jnp.dot(r, g1_ref[...], preferred_element_type=f32))
    Vg = gate * P[:, 0:48]

    P = jnp.dot(Vg, wc2_ref[...], preferred_element_type=f32)
    r = rnorm(P, s1_ref)
    x = jnp.concatenate([se, r], axis=1).astype(bf16)
    se = jnp.dot(x, w2_ref[...], preferred_element_type=f32) + b2_ref[...]
    gate = jnp.dot(r, g1_ref[...], preferred_element_type=f32)
    Vg = gate * P[:, 0:48]

    Vha = jnp.dot(Vg, wha_ref[...], preferred_element_type=f32)
    sha = rnorm(Vha, s16_ref)
    xa = jnp.concatenate([se, sha], axis=1).astype(bf16)
    att = jax.nn.sigmoid(
        jnp.dot(xa, wa_ref[...], preferred_element_type=f32) + ba_ref[...])

    ms_ref[...] = att * se
    mv_ref[:, 0:48] = att * Vg
    mv_ref[:, 48:128] = jnp.zeros((be, 80), f32)


def _edge(gd, gs, es, ev, wts):
    be = 4000
    full = lambda arr: pl.BlockSpec(arr.shape, lambda i: tuple(0 for _ in arr.shape))
    return pl.pallas_call(
        _edge_body,
        grid=(E // be,),
        in_specs=[
            pl.BlockSpec((be, 128), lambda i: (i, 0)),
            pl.BlockSpec((be, 128), lambda i: (i, 0)),
            pl.BlockSpec((be, 32), lambda i: (i, 0)),
            pl.BlockSpec((be, 3), lambda i: (i, 0)),
        ] + [full(w) for w in wts],
        out_specs=[
            pl.BlockSpec((be, 128), lambda i: (i, 0)),
            pl.BlockSpec((be, 128), lambda i: (i, 0)),
        ],
        out_shape=[
            jax.ShapeDtypeStruct((E, 128), jnp.float32),
            jax.ShapeDtypeStruct((E, 128), jnp.float32),
        ],
        compiler_params=pltpu.CompilerParams(
            dimension_semantics=("arbitrary",)),
    )(gd, gs, es, ev, *wts)


def _sc_scatter_body(ms_hbm, mv_hbm, idx_hbm, zeros_hbm, outs_hbm, outv_hbm,
                     acc, idx_v, buf, sem):
    cid = lax.axis_index("c")
    sid = lax.axis_index("s")
    base_r = sid * ROWS_A
    pltpu.sync_copy(zeros_hbm.at[pl.ds(base_r, ROWS_A)],
                    acc.at[pl.ds(base_r, ROWS_A)])

    @pl.when(sid == 0)
    def _():
        pltpu.sync_copy(zeros_hbm.at[pl.ds(16 * ROWS_A, ROWS_TAIL)],
                        acc.at[pl.ds(16 * ROWS_A, ROWS_TAIL)])

    pltpu.sync_copy(idx_hbm.at[sid], idx_v)
    plsc.subcore_barrier()

    def run(m_hbm):
        def body(j, carry):
            base = sid * E_PER_T + j * CHUNK
            pltpu.sync_copy(m_hbm.at[pl.ds(base, CHUNK)], buf)
            pltpu.sync_copy(buf, acc.at[idx_v.at[j]], add=True)
            return carry
        lax.fori_loop(0, NCHUNK_S, body, 0)

    @pl.when(cid == 0)
    def _():
        run(ms_hbm)

    @pl.when(cid == 1)
    def _():
        run(mv_hbm)

    plsc.subcore_barrier()

    def out(out_hbm):
        pltpu.sync_copy(acc.at[pl.ds(base_r, ROWS_A)],
                        out_hbm.at[pl.ds(base_r, ROWS_A)])

        @pl.when(sid == 0)
        def _():
            pltpu.sync_copy(acc.at[pl.ds(16 * ROWS_A, ROWS_TAIL)],
                            out_hbm.at[pl.ds(16 * ROWS_A, ROWS_TAIL)])

    @pl.when(cid == 0)
    def _():
        out(outs_hbm)

    @pl.when(cid == 1)
    def _():
        out(outv_hbm)


def _sc_scatter(ms, mv, idx3, zeros_nrow):
    k = functools.partial(
        pl.kernel,
        out_type=(
            jax.ShapeDtypeStruct((N, 128), jnp.float32),
            jax.ShapeDtypeStruct((N, 128), jnp.float32),
        ),
        mesh=plsc.VectorSubcoreMesh(**_SC_MESH),
        scratch_types=[
            pltpu.VMEM_SHARED((N, 128), jnp.float32),
            pltpu.VMEM((NCHUNK_S, CHUNK), jnp.int32),
            pltpu.VMEM((CHUNK, 128), jnp.float32),
            pltpu.SemaphoreType.DMA,
        ],
    )(_sc_scatter_body)
    return k(ms, mv, idx3, zeros_nrow)


def _node_body(s_ref, vt_ref, as_ref, av_ref,
               fwh0T_ref, fmu0T_ref, f0sT_ref, f0hT_ref, f0b_ref,
               fwh1T_ref, fmu1T_ref, f1sT_ref, f1hT_ref, f1b_ref,
               g0_ref, c0_ref, g1_ref, c1_ref,
               s2_ref, v2_ref):
    f32 = jnp.float32

    def ln(sx, Vx, g, b):
        mu = jnp.mean(sx, axis=-1, keepdims=True)
        var = jnp.mean((sx - mu) ** 2, axis=-1, keepdims=True)
        so = (sx - mu) * jax.lax.rsqrt(var + 1e-5) * g + b
        nrm = jnp.sqrt(jnp.sum(Vx[0] ** 2 + Vx[1] ** 2 + Vx[2] ** 2,
                               axis=-1, keepdims=True)) * 0.25
        nrm = jnp.maximum(nrm, 0.0031622776601683794)
        inv = 1.0 / nrm
        return so, [v * inv for v in Vx]

    s_in = s_ref[...] + as_ref[...]
    V_in = [vt_ref[:, 16 * d:16 * (d + 1)] + av_ref[:, 16 * d:16 * (d + 1)]
            for d in range(3)]
    s1, V1 = ln(s_in, V_in, g0_ref[...], c0_ref[...])

    se, Vg = s1, V1
    for whT, wmuT, wsT, whsT, bb, is_last in (
        (fwh0T_ref, fmu0T_ref, f0sT_ref, f0hT_ref, f0b_ref, False),
        (fwh1T_ref, fmu1T_ref, f1sT_ref, f1hT_ref, f1b_ref, True),
    ):
        Vh = [jnp.dot(v, whT[...], preferred_element_type=f32) for v in Vg]
        sh = jnp.maximum(jnp.sqrt(Vh[0] ** 2 + Vh[1] ** 2 + Vh[2] ** 2), EPS)
        sm = (jnp.dot(se, wsT[...], preferred_element_type=f32)
              + jnp.dot(sh, whsT[...], preferred_element_type=f32)
              + bb[...])
        se = sm if is_last else jax.nn.relu(sm)
        Vmu = [jnp.dot(v, wmuT[...], preferred_element_type=f32) for v in Vh]
        vn = jnp.maximum(jnp.sqrt(Vmu[0] ** 2 + Vmu[1] ** 2 + Vmu[2] ** 2), EPS)
        gate = vn if is_last else jax.nn.sigmoid(vn)
        Vg = [gate * v for v in Vmu]

    s2, V2 = ln(s1 + se, [a + b for a, b in zip(V1, Vg)], g1_ref[...], c1_ref[...])
    s2_ref[...] = s2
    for d in range(3):
        v2_ref[:, 16 * d:16 * (d + 1)] = V2[d]


def _node(s, vt, aggs, aggv, wts):
    bn = 2000
    full = lambda arr: pl.BlockSpec(arr.shape, lambda i: tuple(0 for _ in arr.shape))
    return pl.pallas_call(
        _node_body,
        grid=(N // bn,),
        in_specs=[
            pl.BlockSpec((bn, 128), lambda i: (i, 0)),
            pl.BlockSpec((bn, 48), lambda i: (i, 0)),
            pl.BlockSpec((bn, 128), lambda i: (i, 0)),
            pl.BlockSpec((bn, 128), lambda i: (i, 0)),
        ] + [full(w) for w in wts],
        out_specs=[
            pl.BlockSpec((bn, 128), lambda i: (i, 0)),
            pl.BlockSpec((bn, 48), lambda i: (i, 0)),
        ],
        out_shape=[
            jax.ShapeDtypeStruct((N, 128), jnp.float32),
            jax.ShapeDtypeStruct((N, 48), jnp.float32),
        ],
    )(s, vt, aggs, aggv, *wts)


def _edge_weights(eg0_Wh, eg0_Wmu, eg0_Wm_w, eg0_Wm_b,
                  eg1_Wh, eg1_Wmu, eg1_Wm_w, eg1_Wm_b,
                  eg2_Wh, eg2_Wmu, eg2_Wm_w, eg2_Wm_b,
                  att_Wh, att_Wm_w, att_Wm_b):
    f32 = jnp.float32
    bf16 = jnp.bfloat16
    eye3 = jnp.eye(3, dtype=f32)
    bd = lambda m: jnp.kron(eye3, m)
    sum16 = jnp.kron(jnp.ones((3, 1), f32), jnp.eye(16, dtype=f32))
    exp16 = jnp.kron(jnp.ones((1, 3), f32), jnp.eye(16, dtype=f32))

    w0v = jnp.concatenate(
        [bd(eg0_Wh[:, 0:16].T), bd(eg0_Wh[:, 16:32].T),
         bd(eg0_Wh[:, 32].reshape(1, 33))], axis=0)
    m0bd = bd(eg0_Wmu.T)
    wc0 = jnp.concatenate([jnp.dot(w0v, m0bd), w0v], axis=1)
    s0 = jnp.concatenate([
        jnp.concatenate([jnp.zeros((48, 33), f32), sum16], axis=1),
        jnp.concatenate([jnp.kron(jnp.ones((3, 1), f32),
                                  jnp.eye(33, dtype=f32)),
                         jnp.zeros((99, 16), f32)], axis=1),
    ], axis=0)
    g0 = jnp.concatenate([jnp.zeros((33, 48), f32), exp16], axis=0)
    w0 = jnp.concatenate([eg0_Wm_w[:, 256:321].T,
                          jnp.zeros((16, 128), f32)], axis=0)

    def layer(wh, wmu, wm):
        whbd = bd(wh.T)
        wc = jnp.concatenate([jnp.dot(whbd, bd(wmu.T)), whbd], axis=1)
        w = jnp.concatenate([wm.T, jnp.zeros((16, 128), f32)], axis=0)
        return wc, w

    s1 = jnp.concatenate([
        jnp.concatenate([jnp.zeros((48, 16), f32), sum16], axis=1),
        jnp.concatenate([sum16, jnp.zeros((48, 16), f32)], axis=1),
    ], axis=0)
    g1 = jnp.concatenate([jnp.zeros((16, 48), f32), exp16], axis=0)
    wc1, w1 = layer(eg1_Wh, eg1_Wmu, eg1_Wm_w)
    wc2, w2 = layer(eg2_Wh, eg2_Wmu, eg2_Wm_w)
    return [
        wc0, s0, g0, w0.astype(bf16), eg0_Wm_b.reshape(1, 128),
        wc1, s1, g1, w1.astype(bf16), eg1_Wm_b.reshape(1, 128),
        wc2, w2.astype(bf16), eg2_Wm_b.reshape(1, 128),
        bd(att_Wh.T), sum16, att_Wm_w.T.astype(bf16),
        att_Wm_b.reshape(1, 1),
    ]


def kernel(s, V, edge_index, edge_s, edge_v,
           eg0_Wh, eg0_Wmu, eg0_Wm_w, eg0_Wm_b,
           eg1_Wh, eg1_Wmu, eg1_Wm_w, eg1_Wm_b,
           eg2_Wh, eg2_Wmu, eg2_Wm_w, eg2_Wm_b,
           ff0_Wh, ff0_Wmu, ff0_Wm_w, ff0_Wm_b,
           ff1_Wh, ff1_Wmu, ff1_Wm_w, ff1_Wm_b,
           att_Wh, att_Wm_w, att_Wm_b,
           ln0_g, ln0_b, ln1_g, ln1_b):
    vt = V.transpose(0, 2, 1).reshape(N, 48)
    src = edge_index[0].astype(jnp.int32)
    dst = edge_index[1].astype(jnp.int32)
    idxd3 = dst.reshape(NW, NCHUNK_G, CHUNK)
    idxs3 = src.reshape(NW, NCHUNK_G, CHUNK)
    idxd_sc = dst.reshape(16, NCHUNK_S, CHUNK)
    ev = edge_v.reshape(E, 3)

    td, ts = _prep(s, vt, eg0_Wm_w[:, 0:128].T, eg0_Wm_w[:, 128:256].T)

    gd, gs = _sc_gather(td, ts, idxd3, idxs3)

    edge_wts = _edge_weights(eg0_Wh, eg0_Wmu, eg0_Wm_w, eg0_Wm_b,
                             eg1_Wh, eg1_Wmu, eg1_Wm_w, eg1_Wm_b,
                             eg2_Wh, eg2_Wmu, eg2_Wm_w, eg2_Wm_b,
                             att_Wh, att_Wm_w, att_Wm_b)
    ms, mv = _edge(gd, gs, edge_s, ev, edge_wts)

    zeros_nrow = jnp.zeros((N, 128), jnp.float32)
    aggs, aggv = _sc_scatter(ms, mv, idxd_sc, zeros_nrow)

    node_wts = [
        ff0_Wh.T, ff0_Wmu.T, ff0_Wm_w[:, 0:128].T, ff0_Wm_w[:, 128:144].T,
        ff0_Wm_b.reshape(1, 128),
        ff1_Wh.T, ff1_Wmu.T, ff1_Wm_w[:, 0:128].T, ff1_Wm_w[:, 128:144].T,
        ff1_Wm_b.reshape(1, 128),
        ln0_g.reshape(1, 128), ln0_b.reshape(1, 128),
        ln1_g.reshape(1, 128), ln1_b.reshape(1, 128),
    ]
    s2, v2t = _node(s, vt, aggs, aggv, node_wts)
    V2 = v2t.reshape(N, 3, 16).transpose(0, 2, 1)
    return s2, V2


# speedup vs baseline: 1.8851x; 1.1305x over previous
import functools

import jax
import jax.numpy as jnp
import numpy as np
from jax import lax
from jax.experimental import pallas as pl
from jax.experimental.pallas import tpu as pltpu
from jax.experimental.pallas import tpu_sc as plsc

N = 10000
E = 320000
EPS = 1e-4

NW = 32
E_PER_W = E // NW
CHUNK = 80
NCHUNK_G = E_PER_W // CHUNK
E_PER_T = E // 16
NCHUNK_S = E_PER_T // CHUNK
ROWS_A = 624
ROWS_TAIL = N - 16 * ROWS_A

_SC_MESH = dict(core_axis_name="c", subcore_axis_name="s")
_MASK_HI = np.uint32(0xFFFF0000)
_HALF = np.uint32(0x8000)


def _pack_bf16(x):
    ua = lax.bitcast_convert_type(x[:, 0:64], jnp.uint32)
    ua = (ua + _HALF) & _MASK_HI
    ub = lax.bitcast_convert_type(x[:, 64:128], jnp.uint32)
    ub = (ub + _HALF) >> 16
    return lax.bitcast_convert_type(ua | ub, jnp.float32)


def _unpack_bf16(w):
    u = lax.bitcast_convert_type(w, jnp.uint32)
    hi = lax.bitcast_convert_type(u & _MASK_HI, jnp.float32)
    lo = lax.bitcast_convert_type(u << 16, jnp.float32)
    return jnp.concatenate([hi, lo], axis=1)


def _prep_body(s_ref, vt_ref, wd_ref, ws_ref, td_ref, ts_ref):
    sblk = s_ref[...]
    vt = vt_ref[...]
    zpad = jnp.zeros((sblk.shape[0], 16), jnp.float32)
    for w_ref, out_ref in ((wd_ref, td_ref), (ws_ref, ts_ref)):
        proj = jnp.dot(sblk, w_ref[...], preferred_element_type=jnp.float32)
        out_ref[:, 0:64] = _pack_bf16(proj)
        out_ref[:, 64:112] = vt
        out_ref[:, 112:128] = zpad


def _prep(s, vt, wdT, wsT):
    bn = 2000
    return pl.pallas_call(
        _prep_body,
        grid=(N // bn,),
        in_specs=[
            pl.BlockSpec((bn, 128), lambda i: (i, 0)),
            pl.BlockSpec((bn, 48), lambda i: (i, 0)),
            pl.BlockSpec((128, 128), lambda i: (0, 0)),
            pl.BlockSpec((128, 128), lambda i: (0, 0)),
        ],
        out_specs=[
            pl.BlockSpec((bn, 128), lambda i: (i, 0)),
            pl.BlockSpec((bn, 128), lambda i: (i, 0)),
        ],
        out_shape=[
            jax.ShapeDtypeStruct((N, 128), jnp.float32),
            jax.ShapeDtypeStruct((N, 128), jnp.float32),
        ],
    )(s, vt, wdT, wsT)


def _sc_gather_body(td_hbm, ts_hbm, idxd_hbm, idxs_hbm, gd_hbm, gs_hbm,
                    idxd_v, idxs_v, bufd0, bufd1, bufs0, bufs1,
                    semd0, semd1, sems0, sems1):
    cid = lax.axis_index("c")
    sid = lax.axis_index("s")
    wid = cid * 16 + sid
    pltpu.sync_copy(idxd_hbm.at[wid], idxd_v)
    pltpu.sync_copy(idxs_hbm.at[wid], idxs_v)

    def start(j, bufd, bufs, semd, sems):
        pltpu.async_copy(td_hbm.at[idxd_v.at[j]], bufd, semd)
        pltpu.async_copy(ts_hbm.at[idxs_v.at[j]], bufs, sems)

    def finish(j, bufd, bufs, semd, sems):
        base = wid * E_PER_W + j * CHUNK
        pltpu.make_async_copy(td_hbm.at[idxd_v.at[0]], bufd, semd).wait()
        pltpu.sync_copy(bufd, gd_hbm.at[pl.ds(base, CHUNK)])
        pltpu.make_async_copy(ts_hbm.at[idxs_v.at[0]], bufs, sems).wait()
        pltpu.sync_copy(bufs, gs_hbm.at[pl.ds(base, CHUNK)])

    start(0, bufd0, bufs0, semd0, sems0)
    start(1, bufd1, bufs1, semd1, sems1)

    def body(jo, carry):
        for par, bufd, bufs, semd, sems in (
            (0, bufd0, bufs0, semd0, sems0),
            (1, bufd1, bufs1, semd1, sems1),
        ):
            j = 2 * jo + par
            finish(j, bufd, bufs, semd, sems)

            @pl.when(j + 2 < NCHUNK_G)
            def _():
                start(j + 2, bufd, bufs, semd, sems)
        return carry

    lax.fori_loop(0, NCHUNK_G // 2, body, 0)
    finish(NCHUNK_G - 1, bufd0, bufs0, semd0, sems0)


def _sc_gather(td, ts, idxd3, idxs3):
    k = functools.partial(
        pl.kernel,
        out_type=(
            jax.ShapeDtypeStruct((E, 128), jnp.float32),
            jax.ShapeDtypeStruct((E, 128), jnp.float32),
        ),
        mesh=plsc.VectorSubcoreMesh(**_SC_MESH),
        scratch_types=[
            pltpu.VMEM((NCHUNK_G, CHUNK), jnp.int32),
            pltpu.VMEM((NCHUNK_G, CHUNK), jnp.int32),
            pltpu.VMEM((CHUNK, 128), jnp.float32),
            pltpu.VMEM((CHUNK, 128), jnp.float32),
            pltpu.VMEM((CHUNK, 128), jnp.float32),
            pltpu.VMEM((CHUNK, 128), jnp.float32),
            pltpu.SemaphoreType.DMA,
            pltpu.SemaphoreType.DMA,
            pltpu.SemaphoreType.DMA,
            pltpu.SemaphoreType.DMA,
        ],
    )(_sc_gather_body)
    return k(td, ts, idxd3, idxs3)


_EPS2 = float(np.float32(EPS) * np.float32(EPS))


def _edge_body(gd_ref, gs_ref, es_ref, ev_ref,
               wc0_ref, s0_ref, g0_ref, w0_ref, b0_ref,
               wc1_ref, s1_ref, g1_ref, w1_ref, b1_ref,
               wc2_ref, w2_ref, b2_ref,
               wha_ref, s16_ref, wa_ref, ba_ref,
               ms_ref, mv_ref):
    f32 = jnp.float32
    bf16 = jnp.bfloat16
    gd = gd_ref[...]
    gs = gs_ref[...]
    be = gd.shape[0]

    def rnorm(p, s_ref):
        xc = jnp.maximum(
            jnp.dot(p * p, s_ref[...], preferred_element_type=f32), _EPS2)
        return jnp.sqrt(xc)

    x0v = jnp.concatenate([gd[:, 64:112], gs[:, 64:112], ev_ref[...]], axis=1)
    P = jnp.dot(x0v, wc0_ref[...], preferred_element_type=f32)
    r = rnorm(P, s0_ref)
    x0 = jnp.concatenate([es_ref[...], r], axis=1).astype(bf16)
    sm = (_unpack_bf16(gd[:, 0:64]) + _unpack_bf16(gs[:, 0:64])
          + jnp.dot(x0, w0_ref[...], preferred_element_type=f32) + b0_ref[...])
    se = jax.nn.relu(sm)
    gate = jax.nn.sigmoid(jnp.dot(r, g0_ref[...], preferred_element_type=f32))
    Vg = gate * P[:, 0:48]

    P = jnp.dot(Vg, wc1_ref[...], preferred_element_type=f32)
    r = rnorm(P, s1_ref)
    x = jnp.concatenate([se, r], axis=1).astype(bf16)
    se = jax.nn.relu(
        jnp.dot(x, w1_ref[...], preferred_element_type=f32) + b1_ref[...])
    gate = jax.nn.sigmoid(jnp.dot(r, g1_ref[...], preferred_element_type=f32))
    Vg = gate * P[:, 0:48]

    P = jnp.dot(Vg, wc2_ref[...], preferred_element_type=f32)
    r = rnorm(P, s1_ref)
    x = jnp.concatenate([se, r], axis=1).astype(bf16)
    se = jnp.dot(x, w2_ref[...], preferred_element_type=f32) + b2_ref[...]
    gate = jnp.dot(r, g1_ref[...], preferred_element_type=f32)
    Vg = gate * P[:, 0:48]

    Vha = jnp.dot(Vg, wha_ref[...], preferred_element_type=f32)
    sha = rnorm(Vha, s16_ref)
    xa = jnp.concatenate([se, sha], axis=1).astype(bf16)
    att = jax.nn.sigmoid(
        jnp.dot(xa, wa_ref[...], preferred_element_type=f32) + ba_ref[...])

    ms_ref[...] = att * se
    mv_ref[:, 0:48] = att * Vg
    mv_ref[:, 48:128] = jnp.zeros((be, 80), f32)


def _edge(gd, gs, es, ev, wts):
    be = 4000
    full = lambda arr: pl.BlockSpec(arr.shape, lambda i: tuple(0 for _ in arr.shape))
    return pl.pallas_call(
        _edge_body,
        grid=(E // be,),
        in_specs=[
            pl.BlockSpec((be, 128), lambda i: (i, 0)),
            pl.BlockSpec((be, 128), lambda i: (i, 0)),
            pl.BlockSpec((be, 32), lambda i: (i, 0)),
            pl.BlockSpec((be, 3), lambda i: (i, 0)),
        ] + [full(w) for w in wts],
        out_specs=[
            pl.BlockSpec((be, 128), lambda i: (i, 0)),
            pl.BlockSpec((be, 128), lambda i: (i, 0)),
        ],
        out_shape=[
            jax.ShapeDtypeStruct((E, 128), jnp.float32),
            jax.ShapeDtypeStruct((E, 128), jnp.float32),
        ],
        compiler_params=pltpu.CompilerParams(
            dimension_semantics=("arbitrary",)),
    )(gd, gs, es, ev, *wts)


def _sc_scatter_body(ms_hbm, mv_hbm, idx_hbm, zeros_hbm, outs_hbm, outv_hbm,
                     acc, idx2, buf0, buf1, sem0, sem1, semi0, semi1):
    cid = lax.axis_index("c")
    sid = lax.axis_index("s")
    base_r = sid * ROWS_A
    pltpu.sync_copy(zeros_hbm.at[pl.ds(base_r, ROWS_A)],
                    acc.at[pl.ds(base_r, ROWS_A)])

    @pl.when(sid == 0)
    def _():
        pltpu.sync_copy(zeros_hbm.at[pl.ds(16 * ROWS_A, ROWS_TAIL)],
                        acc.at[pl.ds(16 * ROWS_A, ROWS_TAIL)])

    plsc.subcore_barrier()

    def run(m_hbm):
        def start(j, par, buf, sem, semi):
            base = sid * E_PER_T + j * CHUNK
            pltpu.async_copy(m_hbm.at[pl.ds(base, CHUNK)], buf, sem)
            pltpu.async_copy(idx_hbm.at[sid].at[j], idx2.at[par], semi)

        start(0, 0, buf0, sem0, semi0)
        start(1, 1, buf1, sem1, semi1)

        def body(jo, carry):
            for par, buf, sem, semi in ((0, buf0, sem0, semi0),
                                        (1, buf1, sem1, semi1)):
                j = 2 * jo + par
                pltpu.make_async_copy(m_hbm.at[pl.ds(0, CHUNK)], buf, sem).wait()
                pltpu.make_async_copy(idx_hbm.at[sid].at[0], idx2.at[par],
                                      semi).wait()
                pltpu.sync_copy(buf, acc.at[idx2.at[par]], add=True)

                @pl.when(j + 2 < NCHUNK_S)
                def _():
                    start(j + 2, par, buf, sem, semi)
            return carry

        lax.fori_loop(0, NCHUNK_S // 2, body, 0)

    @pl.when(cid == 0)
    def _():
        run(ms_hbm)

    @pl.when(cid == 1)
    def _():
        run(mv_hbm)

    plsc.subcore_barrier()

    def out(out_hbm):
        pltpu.sync_copy(acc.at[pl.ds(base_r, ROWS_A)],
                        out_hbm.at[pl.ds(base_r, ROWS_A)])

        @pl.when(sid == 0)
        def _():
            pltpu.sync_copy(acc.at[pl.ds(16 * ROWS_A, ROWS_TAIL)],
                            out_hbm.at[pl.ds(16 * ROWS_A, ROWS_TAIL)])

    @pl.when(cid == 0)
    def _():
        out(outs_hbm)

    @pl.when(cid == 1)
    def _():
        out(outv_hbm)


def _sc_scatter(ms, mv, idx3, zeros_nrow):
    k = functools.partial(
        pl.kernel,
        out_type=(
            jax.ShapeDtypeStruct((N, 128), jnp.float32),
            jax.ShapeDtypeStruct((N, 128), jnp.float32),
        ),
        mesh=plsc.VectorSubcoreMesh(**_SC_MESH),
        scratch_types=[
            pltpu.VMEM_SHARED((N, 128), jnp.float32),
            pltpu.VMEM((2, CHUNK), jnp.int32),
            pltpu.VMEM((CHUNK, 128), jnp.float32),
            pltpu.VMEM((CHUNK, 128), jnp.float32),
            pltpu.SemaphoreType.DMA,
            pltpu.SemaphoreType.DMA,
            pltpu.SemaphoreType.DMA,
            pltpu.SemaphoreType.DMA,
        ],
    )(_sc_scatter_body)
    return k(ms, mv, idx3, zeros_nrow)


def _node_body(s_ref, vt_ref, as_ref, av_ref,
               fwh0T_ref, fmu0T_ref, f0sT_ref, f0hT_ref, f0b_ref,
               fwh1T_ref, fmu1T_ref, f1sT_ref, f1hT_ref, f1b_ref,
               g0_ref, c0_ref, g1_ref, c1_ref,
               s2_ref, v2_ref):
    f32 = jnp.float32

    def ln(sx, Vx, g, b):
        mu = jnp.mean(sx, axis=-1, keepdims=True)
        var = jnp.mean((sx - mu) ** 2, axis=-1, keepdims=True)
        so = (sx - mu) * jax.lax.rsqrt(var + 1e-5) * g + b
        nrm = jnp.sqrt(jnp.sum(Vx[0] ** 2 + Vx[1] ** 2 + Vx[2] ** 2,
                               axis=-1, keepdims=True)) * 0.25
        nrm = jnp.maximum(nrm, 0.0031622776601683794)
        inv = 1.0 / nrm
        return so, [v * inv for v in Vx]

    s_in = s_ref[...] + as_ref[...]
    V_in = [vt_ref[:, 16 * d:16 * (d + 1)] + av_ref[:, 16 * d:16 * (d + 1)]
            for d in range(3)]
    s1, V1 = ln(s_in, V_in, g0_ref[...], c0_ref[...])

    se, Vg = s1, V1
    for whT, wmuT, wsT, whsT, bb, is_last in (
        (fwh0T_ref, fmu0T_ref, f0sT_ref, f0hT_ref, f0b_ref, False),
        (fwh1T_ref, fmu1T_ref, f1sT_ref, f1hT_ref, f1b_ref, True),
    ):
        Vh = [jnp.dot(v, whT[...], preferred_element_type=f32) for v in Vg]
        sh = jnp.maximum(jnp.sqrt(Vh[0] ** 2 + Vh[1] ** 2 + Vh[2] ** 2), EPS)
        sm = (jnp.dot(se, wsT[...], preferred_element_type=f32)
              + jnp.dot(sh, whsT[...], preferred_element_type=f32)
              + bb[...])
        se = sm if is_last else jax.nn.relu(sm)
        Vmu = [jnp.dot(v, wmuT[...], preferred_element_type=f32) for v in Vh]
        vn = jnp.maximum(jnp.sqrt(Vmu[0] ** 2 + Vmu[1] ** 2 + Vmu[2] ** 2), EPS)
        gate = vn if is_last else jax.nn.sigmoid(vn)
        Vg = [gate * v for v in Vmu]

    s2, V2 = ln(s1 + se, [a + b for a, b in zip(V1, Vg)], g1_ref[...], c1_ref[...])
    s2_ref[...] = s2
    for d in range(3):
        v2_ref[:, 16 * d:16 * (d + 1)] = V2[d]


def _node(s, vt, aggs, aggv, wts):
    bn = 2000
    full = lambda arr: pl.BlockSpec(arr.shape, lambda i: tuple(0 for _ in arr.shape))
    return pl.pallas_call(
        _node_body,
        grid=(N // bn,),
        in_specs=[
            pl.BlockSpec((bn, 128), lambda i: (i, 0)),
            pl.BlockSpec((bn, 48), lambda i: (i, 0)),
            pl.BlockSpec((bn, 128), lambda i: (i, 0)),
            pl.BlockSpec((bn, 128), lambda i: (i, 0)),
        ] + [full(w) for w in wts],
        out_specs=[
            pl.BlockSpec((bn, 128), lambda i: (i, 0)),
            pl.BlockSpec((bn, 48), lambda i: (i, 0)),
        ],
        out_shape=[
            jax.ShapeDtypeStruct((N, 128), jnp.float32),
            jax.ShapeDtypeStruct((N, 48), jnp.float32),
        ],
    )(s, vt, aggs, aggv, *wts)


def _edge_weights(eg0_Wh, eg0_Wmu, eg0_Wm_w, eg0_Wm_b,
                  eg1_Wh, eg1_Wmu, eg1_Wm_w, eg1_Wm_b,
                  eg2_Wh, eg2_Wmu, eg2_Wm_w, eg2_Wm_b,
                  att_Wh, att_Wm_w, att_Wm_b):
    f32 = jnp.float32
    bf16 = jnp.bfloat16
    eye3 = jnp.eye(3, dtype=f32)
    bd = lambda m: jnp.kron(eye3, m)
    sum16 = jnp.kron(jnp.ones((3, 1), f32), jnp.eye(16, dtype=f32))
    exp16 = jnp.kron(jnp.ones((1, 3), f32), jnp.eye(16, dtype=f32))

    w0v = jnp.concatenate(
        [bd(eg0_Wh[:, 0:16].T), bd(eg0_Wh[:, 16:32].T),
         bd(eg0_Wh[:, 32].reshape(1, 33))], axis=0)
    m0bd = bd(eg0_Wmu.T)
    wc0 = jnp.concatenate([jnp.dot(w0v, m0bd), w0v], axis=1)
    s0 = jnp.concatenate([
        jnp.concatenate([jnp.zeros((48, 33), f32), sum16], axis=1),
        jnp.concatenate([jnp.kron(jnp.ones((3, 1), f32),
                                  jnp.eye(33, dtype=f32)),
                         jnp.zeros((99, 16), f32)], axis=1),
    ], axis=0)
    g0 = jnp.concatenate([jnp.zeros((33, 48), f32), exp16], axis=0)
    w0 = jnp.concatenate([eg0_Wm_w[:, 256:321].T,
                          jnp.zeros((16, 128), f32)], axis=0)

    def layer(wh, wmu, wm):
        whbd = bd(wh.T)
        wc = jnp.concatenate([jnp.dot(whbd, bd(wmu.T)), whbd], axis=1)
        w = jnp.concatenate([wm.T, jnp.zeros((16, 128), f32)], axis=0)
        return wc, w

    s1 = jnp.concatenate([
        jnp.concatenate([jnp.zeros((48, 16), f32), sum16], axis=1),
        jnp.concatenate([sum16, jnp.zeros((48, 16), f32)], axis=1),
    ], axis=0)
    g1 = jnp.concatenate([jnp.zeros((16, 48), f32), exp16], axis=0)
    wc1, w1 = layer(eg1_Wh, eg1_Wmu, eg1_Wm_w)
    wc2, w2 = layer(eg2_Wh, eg2_Wmu, eg2_Wm_w)
    return [
        wc0, s0, g0, w0.astype(bf16), eg0_Wm_b.reshape(1, 128),
        wc1, s1, g1, w1.astype(bf16), eg1_Wm_b.reshape(1, 128),
        wc2, w2.astype(bf16), eg2_Wm_b.reshape(1, 128),
        bd(att_Wh.T), sum16, att_Wm_w.T.astype(bf16),
        att_Wm_b.reshape(1, 1),
    ]


def kernel(s, V, edge_index, edge_s, edge_v,
           eg0_Wh, eg0_Wmu, eg0_Wm_w, eg0_Wm_b,
           eg1_Wh, eg1_Wmu, eg1_Wm_w, eg1_Wm_b,
           eg2_Wh, eg2_Wmu, eg2_Wm_w, eg2_Wm_b,
           ff0_Wh, ff0_Wmu, ff0_Wm_w, ff0_Wm_b,
           ff1_Wh, ff1_Wmu, ff1_Wm_w, ff1_Wm_b,
           att_Wh, att_Wm_w, att_Wm_b,
           ln0_g, ln0_b, ln1_g, ln1_b):
    vt = V.transpose(0, 2, 1).reshape(N, 48)
    src = edge_index[0].astype(jnp.int32)
    dst = edge_index[1].astype(jnp.int32)
    idxd3 = dst.reshape(NW, NCHUNK_G, CHUNK)
    idxs3 = src.reshape(NW, NCHUNK_G, CHUNK)
    idxd_sc = dst.reshape(16, NCHUNK_S, CHUNK)
    ev = edge_v.reshape(E, 3)

    td, ts = _prep(s, vt, eg0_Wm_w[:, 0:128].T, eg0_Wm_w[:, 128:256].T)

    gd, gs = _sc_gather(td, ts, idxd3, idxs3)

    edge_wts = _edge_weights(eg0_Wh, eg0_Wmu, eg0_Wm_w, eg0_Wm_b,
                             eg1_Wh, eg1_Wmu, eg1_Wm_w, eg1_Wm_b,
                             eg2_Wh, eg2_Wmu, eg2_Wm_w, eg2_Wm_b,
                             att_Wh, att_Wm_w, att_Wm_b)
    ms, mv = _edge(gd, gs, edge_s, ev, edge_wts)

    zeros_nrow = jnp.zeros((N, 128), jnp.float32)
    aggs, aggv = _sc_scatter(ms, mv, idxd_sc, zeros_nrow)

    node_wts = [
        ff0_Wh.T, ff0_Wmu.T, ff0_Wm_w[:, 0:128].T, ff0_Wm_w[:, 128:144].T,
        ff0_Wm_b.reshape(1, 128),
        ff1_Wh.T, ff1_Wmu.T, ff1_Wm_w[:, 0:128].T, ff1_Wm_w[:, 128:144].T,
        ff1_Wm_b.reshape(1, 128),
        ln0_g.reshape(1, 128), ln0_b.reshape(1, 128),
        ln1_g.reshape(1, 128), ln1_b.reshape(1, 128),
    ]
    s2, v2t = _node(s, vt, aggs, aggv, node_wts)
    V2 = v2t.reshape(N, 3, 16).transpose(0, 2, 1)
    return s2, V2
